# baseline (speedup 1.0000x reference)
import functools

import jax
import jax.numpy as jnp
from jax import lax
from jax.experimental import pallas as pl
from jax.experimental.pallas import tpu as pltpu
from jax.experimental.pallas import tpu_sc as plsc

N = 10000
NPAD = 10240
E = 320000
EPT = E // 16
LANES = 16
RNG = NPAD // 16
NCH = RNG // 128

_MESH = plsc.VectorSubcoreMesh(core_axis_name="c", subcore_axis_name="s")
_SC_PARAMS = pltpu.CompilerParams(needs_layout_passes=False,
                                  skip_device_barrier=True)

_W2_OFF = 0
_W3_OFF = 16
_B1_OFF = 24
_B2_OFF = 28
_B3_OFF = 32
_WC_OFF = 34
_BC_OFF = 66
_WT_LEN = 96


def _tc_prep(x, W1, ei):
    def body(x_ref, w1_ref, ei_ref, h0_ref, pk_ref):
        h0t = lax.dot_general(
            w1_ref[...], x_ref[...], (((0,), (1,)), ((), ())),
            preferred_element_type=jnp.float32)
        h0_ref[...] = jnp.zeros_like(h0_ref)
        h0_ref[:, :N] = h0t
        pk_ref[...] = ei_ref[0] * 16384 + ei_ref[1]

    return pl.pallas_call(
        body,
        out_shape=(
            jax.ShapeDtypeStruct((4, NPAD), jnp.float32),
            jax.ShapeDtypeStruct((16, EPT), jnp.int32),
        ),
    )(x, W1, ei)


def _rsqrt16(d):
    i = plsc.bitcast(d, jnp.int32)
    i = 0x5F3759DF - lax.shift_right_logical(i, 1)
    y = plsc.bitcast(i, jnp.float32)
    for _ in range(3):
        y = y * (1.5 - 0.5 * d * y * y)
    return y


def _tanh16(z):
    e = jnp.exp(z + z)
    return 1.0 - 2.0 / (e + 1.0)


@functools.partial(
    pl.kernel,
    out_type=(
        jax.ShapeDtypeStruct((NPAD * 2,), jnp.float32),
        jax.ShapeDtypeStruct((NPAD * 16,), jnp.float32),
    ),
    mesh=_MESH,
    compiler_params=_SC_PARAMS,
    scratch_types=[
        pltpu.VMEM((4, NPAD), jnp.float32),
        pltpu.VMEM((4, NPAD), jnp.float32),
        pltpu.VMEM((EPT,), jnp.int32),
        pltpu.VMEM((_WT_LEN,), jnp.float32),
        pltpu.VMEM((4, RNG), jnp.float32),
        pltpu.VMEM((4, RNG), jnp.float32),
        pltpu.VMEM((RNG,), jnp.float32),
        pltpu.VMEM((16, 4, 128), jnp.float32),
        pltpu.VMEM((16, 128), jnp.float32),
        pltpu.VMEM((RNG * 16,), jnp.float32),
        pltpu.VMEM((RNG * 2,), jnp.float32),
        pltpu.HBM((2, 16, 4, NPAD), jnp.float32),
        pltpu.HBM((2, 4, NPAD), jnp.float32),
        pltpu.SemaphoreType.DMA,
        pltpu.SemaphoreType.DMA,
        pltpu.SemaphoreType.DMA,
    ],
)
def _gcn_sc(h0_hbm, pk_hbm, wt_hbm, h3_hbm, out_hbm, hs_v, acc_v, pk_v, wt_v,
            stage4, agg_rng, dis_rng, red_v, redd, obuf, h3buf, slots, hs_sh,
            sem1, sem2, sem3):
    cid = lax.axis_index("c")
    sid = lax.axis_index("s")
    rng0 = pl.multiple_of(sid * RNG, 128)

    cp_pk = pltpu.async_copy(pk_hbm.at[sid], pk_v, sem1)
    cp_h0 = pltpu.async_copy(h0_hbm.at[:, pl.ds(rng0, RNG)], stage4, sem2)
    cp_wt = pltpu.async_copy(wt_hbm, wt_v, sem3)

    zero = jnp.zeros((LANES,), jnp.float32)
    ones = jnp.ones((LANES,), jnp.float32)
    fidx = [jnp.full((LANES,), f, jnp.int32) for f in range(4)]
    mask = jnp.full((LANES,), 16383, jnp.int32)

    @functools.partial(plsc.parallel_loop, 0, NPAD // LANES, unroll=8)
    def _(i):
        acc_v[0, pl.ds(i * LANES, LANES)] = zero

    cp_pk.wait()

    @functools.partial(plsc.parallel_loop, 0, EPT // LANES, unroll=8)
    def _(i):
        p = pk_v[pl.ds(i * LANES, LANES)]
        c = lax.bitwise_and(p, mask)
        plsc.addupdate_scatter(acc_v, [fidx[0], c], ones)

    pltpu.sync_copy(acc_v.at[0], slots.at[cid, sid, 0])
    plsc.subcore_barrier()

    @pl.loop(0, NCH)
    def _(ch):
        off = pl.multiple_of(rng0 + ch * 128, 128)
        pltpu.sync_copy(slots.at[cid, :, 0, pl.ds(off, 128)], redd)
        for i in range(8):
            sl = pl.ds(i * LANES, LANES)
            a = redd[0, sl]
            for k in range(1, 16):
                a = a + redd[k, sl]
            dis_rng[pl.ds(ch * 128 + i * LANES, LANES)] = _rsqrt16(a + 1.0)

    cp_h0.wait()
    cp_wt.wait()

    @pl.loop(0, RNG // LANES)
    def _(i):
        sl = pl.ds(i * LANES, LANES)
        d = dis_rng[sl]
        for f in range(4):
            stage4[f, sl] = stage4[f, sl] * d

    pltpu.sync_copy(stage4, hs_sh.at[cid, :, pl.ds(rng0, RNG)])
    plsc.subcore_barrier()
    pltpu.sync_copy(hs_sh.at[cid], hs_v)
    plsc.subcore_barrier()

    wchunk = [wt_v[pl.ds(16 * k, 16)] for k in range(_WT_LEN // 16)]

    def wscal(i):
        return jnp.broadcast_to(wchunk[i // 16][i % 16], (LANES,))

    def layer(F, w_off, f_out, b_off, last):
        @functools.partial(plsc.parallel_loop, 0, NPAD // LANES, unroll=8)
        def _(i):
            for f in range(F):
                acc_v[f, pl.ds(i * LANES, LANES)] = zero

        @functools.partial(plsc.parallel_loop, 0, EPT // LANES, unroll=8)
        def _(i):
            p = pk_v[pl.ds(i * LANES, LANES)]
            r = lax.shift_right_logical(p, 14)
            c = lax.bitwise_and(p, mask)
            for f in range(F):
                v = plsc.load_gather(hs_v, [fidx[f], r])
                plsc.addupdate_scatter(acc_v, [fidx[f], c], v)

        pltpu.sync_copy(acc_v.at[pl.ds(0, F)], slots.at[cid, sid, pl.ds(0, F)])
        plsc.subcore_barrier()

        @pl.loop(0, NCH)
        def _(ch):
            off = pl.multiple_of(rng0 + ch * 128, 128)
            pltpu.sync_copy(slots.at[cid, :, pl.ds(0, F), pl.ds(off, 128)],
                            red_v.at[:, pl.ds(0, F)])
            for i in range(8):
                sl = pl.ds(i * LANES, LANES)
                dst = pl.ds(ch * 128 + i * LANES, LANES)
                for f in range(F):
                    a = red_v[0, f, sl]
                    for k in range(1, 16):
                        a = a + red_v[k, f, sl]
                    agg_rng[f, dst] = a

        bvec = [wscal(b_off + f) for f in range(F)]
        if not last:
            wvec = [[wscal(w_off + k * f_out + j) for k in range(F)]
                    for j in range(f_out)]
        else:
            wvec = [[wscal(_WC_OFF + k * 16 + j) for k in range(F)]
                    for j in range(16)]
            bcv = [wscal(_BC_OFF + j) for j in range(16)]

        @pl.loop(0, RNG // LANES)
        def _(i):
            sl = pl.ds(i * LANES, LANES)
            gsl = pl.ds(rng0 + i * LANES, LANES)
            d = dis_rng[sl]
            t = [_tanh16((agg_rng[f, sl] + hs_v[f, gsl]) * d + bvec[f])
                 for f in range(F)]
            if not last:
                for j in range(f_out):
                    s = t[0] * wvec[j][0]
                    for k in range(1, F):
                        s = s + t[k] * wvec[j][k]
                    stage4[j, sl] = s * d
            else:
                lane = jnp.arange(LANES, dtype=jnp.int32)
                n2 = lane * 2 + i * (LANES * 2)
                n16 = lane * 16 + i * (LANES * 16)
                for f in range(F):
                    plsc.store_scatter(h3buf, [n2 + f], t[f])
                for j in range(16):
                    s = t[0] * wvec[j][0]
                    for k in range(1, F):
                        s = s + t[k] * wvec[j][k]
                    plsc.store_scatter(obuf, [n16 + j], s + bcv[j])

        if not last:
            pltpu.sync_copy(stage4.at[pl.ds(0, f_out)],
                            hs_sh.at[cid, pl.ds(0, f_out), pl.ds(rng0, RNG)])
            plsc.subcore_barrier()
            pltpu.sync_copy(hs_sh.at[cid, pl.ds(0, f_out)],
                            hs_v.at[pl.ds(0, f_out)])
            plsc.subcore_barrier()
        else:
            @pl.when(cid == 0)
            def _():
                o2 = pl.multiple_of(sid * (RNG * 2), 128)
                o16 = pl.multiple_of(sid * (RNG * 16), 128)
                pltpu.sync_copy(h3buf, h3_hbm.at[pl.ds(o2, RNG * 2)])
                pltpu.sync_copy(obuf, out_hbm.at[pl.ds(o16, RNG * 16)])

    layer(4, _W2_OFF, 4, _B1_OFF, False)
    layer(4, _W3_OFF, 2, _B2_OFF, False)
    layer(2, 0, 16, _B3_OFF, True)


def kernel(x, edge_index, W1, b1, W2, b2, W3, b3, Wc, bc):
    ei3 = edge_index.reshape(2, 16, EPT)
    h0t, pk = _tc_prep(x, W1, ei3)
    wt = jnp.concatenate([
        W2.ravel(), W3.ravel(), b1, b2, b3, Wc.ravel(), bc,
        jnp.zeros((_WT_LEN - 82,), jnp.float32),
    ])
    h3f, outf = _gcn_sc(h0t, pk, wt)
    return outf.reshape(NPAD, 16)[:N], h3f.reshape(NPAD, 2)[:N]


# speedup vs baseline: 1.0042x; 1.0042x over previous
import functools

import jax
import jax.numpy as jnp
from jax import lax
from jax.experimental import pallas as pl
from jax.experimental.pallas import tpu as pltpu
from jax.experimental.pallas import tpu_sc as plsc

N = 10000
NPAD = 10240
E = 320000
EPT = E // 16
LANES = 16
RNG = NPAD // 16
NCH = RNG // 128

_MESH = plsc.VectorSubcoreMesh(core_axis_name="c", subcore_axis_name="s")
_SC_PARAMS = pltpu.CompilerParams(needs_layout_passes=False,
                                  skip_device_barrier=True)

_W2_OFF = 0
_W3_OFF = 16
_B1_OFF = 24
_B2_OFF = 28
_B3_OFF = 32
_WC_OFF = 34
_BC_OFF = 66
_WT_LEN = 96


def _tc_prep(x, W1, ei):
    def body(x_ref, w1_ref, ei_ref, h0_ref, pk_ref):
        h0t = lax.dot_general(
            w1_ref[...], x_ref[...], (((0,), (1,)), ((), ())),
            preferred_element_type=jnp.float32)
        h0_ref[...] = jnp.zeros_like(h0_ref)
        h0_ref[:, :N] = h0t
        pk_ref[...] = ei_ref[0] * 16384 + ei_ref[1]

    return pl.pallas_call(
        body,
        out_shape=(
            jax.ShapeDtypeStruct((4, NPAD), jnp.float32),
            jax.ShapeDtypeStruct((16, EPT), jnp.int32),
        ),
    )(x, W1, ei)


def _rsqrt16(d):
    i = plsc.bitcast(d, jnp.int32)
    i = 0x5F3759DF - lax.shift_right_logical(i, 1)
    y = plsc.bitcast(i, jnp.float32)
    for _ in range(3):
        y = y * (1.5 - 0.5 * d * y * y)
    return y


def _tanh16(z):
    e = jnp.exp(z + z)
    return 1.0 - 2.0 / (e + 1.0)


@functools.partial(
    pl.kernel,
    out_type=(
        jax.ShapeDtypeStruct((NPAD * 2,), jnp.float32),
        jax.ShapeDtypeStruct((NPAD * 16,), jnp.float32),
    ),
    mesh=_MESH,
    compiler_params=_SC_PARAMS,
    scratch_types=[
        pltpu.VMEM((4, NPAD), jnp.float32),
        pltpu.VMEM((4, NPAD), jnp.float32),
        pltpu.VMEM((EPT,), jnp.int32),
        pltpu.VMEM((_WT_LEN,), jnp.float32),
        pltpu.VMEM((4, RNG), jnp.float32),
        pltpu.VMEM((4, RNG), jnp.float32),
        pltpu.VMEM((RNG,), jnp.float32),
        pltpu.VMEM((16, 4, 128), jnp.float32),
        pltpu.VMEM((16, 128), jnp.float32),
        pltpu.VMEM((RNG * 16,), jnp.float32),
        pltpu.VMEM((RNG * 2,), jnp.float32),
        pltpu.HBM((2, 16, 4, NPAD), jnp.float32),
        pltpu.HBM((2, 4, NPAD), jnp.float32),
        pltpu.SemaphoreType.DMA,
        pltpu.SemaphoreType.DMA,
        pltpu.SemaphoreType.DMA,
    ],
)
def _gcn_sc(h0_hbm, pk_hbm, wt_hbm, h3_hbm, out_hbm, hs_v, acc_v, pk_v, wt_v,
            stage4, agg_rng, dis_rng, red_v, redd, obuf, h3buf, slots, hs_sh,
            sem1, sem2, sem3):
    cid = lax.axis_index("c")
    sid = lax.axis_index("s")
    rng0 = pl.multiple_of(sid * RNG, 128)

    cp_pk = pltpu.async_copy(pk_hbm.at[sid], pk_v, sem1)
    cp_h0 = pltpu.async_copy(h0_hbm.at[:, pl.ds(rng0, RNG)], stage4, sem2)
    cp_wt = pltpu.async_copy(wt_hbm, wt_v, sem3)

    zero = jnp.zeros((LANES,), jnp.float32)
    ones = jnp.ones((LANES,), jnp.float32)
    fidx = [jnp.full((LANES,), f, jnp.int32) for f in range(4)]
    mask = jnp.full((LANES,), 16383, jnp.int32)

    @functools.partial(plsc.parallel_loop, 0, NPAD // LANES, unroll=8)
    def _(i):
        acc_v[0, pl.ds(i * LANES, LANES)] = zero

    cp_pk.wait()

    @functools.partial(plsc.parallel_loop, 0, EPT // LANES, unroll=4)
    def _(i):
        p = pk_v[pl.ds(i * LANES, LANES)]
        c = lax.bitwise_and(p, mask)
        plsc.addupdate_scatter(acc_v, [fidx[0], c], ones)

    pltpu.sync_copy(acc_v.at[0], slots.at[cid, sid, 0])
    plsc.subcore_barrier()

    @pl.loop(0, NCH)
    def _(ch):
        off = pl.multiple_of(rng0 + ch * 128, 128)
        pltpu.sync_copy(slots.at[cid, :, 0, pl.ds(off, 128)], redd)
        for i in range(8):
            sl = pl.ds(i * LANES, LANES)
            a = redd[0, sl]
            for k in range(1, 16):
                a = a + redd[k, sl]
            dis_rng[pl.ds(ch * 128 + i * LANES, LANES)] = _rsqrt16(a + 1.0)

    cp_h0.wait()
    cp_wt.wait()

    @pl.loop(0, RNG // LANES)
    def _(i):
        sl = pl.ds(i * LANES, LANES)
        d = dis_rng[sl]
        for f in range(4):
            stage4[f, sl] = stage4[f, sl] * d

    pltpu.sync_copy(stage4, hs_sh.at[cid, :, pl.ds(rng0, RNG)])
    plsc.subcore_barrier()
    pltpu.sync_copy(hs_sh.at[cid], hs_v)
    plsc.subcore_barrier()

    wchunk = [wt_v[pl.ds(16 * k, 16)] for k in range(_WT_LEN // 16)]

    def wscal(i):
        return jnp.broadcast_to(wchunk[i // 16][i % 16], (LANES,))

    def layer(F, w_off, f_out, b_off, last):
        @functools.partial(plsc.parallel_loop, 0, NPAD // LANES, unroll=8)
        def _(i):
            for f in range(F):
                acc_v[f, pl.ds(i * LANES, LANES)] = zero

        @functools.partial(plsc.parallel_loop, 0, EPT // LANES, unroll=4)
        def _(i):
            p = pk_v[pl.ds(i * LANES, LANES)]
            r = lax.shift_right_logical(p, 14)
            c = lax.bitwise_and(p, mask)
            for f in range(F):
                v = plsc.load_gather(hs_v, [fidx[f], r])
                plsc.addupdate_scatter(acc_v, [fidx[f], c], v)

        pltpu.sync_copy(acc_v.at[pl.ds(0, F)], slots.at[cid, sid, pl.ds(0, F)])
        plsc.subcore_barrier()

        @pl.loop(0, NCH)
        def _(ch):
            off = pl.multiple_of(rng0 + ch * 128, 128)
            pltpu.sync_copy(slots.at[cid, :, pl.ds(0, F), pl.ds(off, 128)],
                            red_v.at[:, pl.ds(0, F)])
            for i in range(8):
                sl = pl.ds(i * LANES, LANES)
                dst = pl.ds(ch * 128 + i * LANES, LANES)
                for f in range(F):
                    a = red_v[0, f, sl]
                    for k in range(1, 16):
                        a = a + red_v[k, f, sl]
                    agg_rng[f, dst] = a

        bvec = [wscal(b_off + f) for f in range(F)]
        if not last:
            wvec = [[wscal(w_off + k * f_out + j) for k in range(F)]
                    for j in range(f_out)]
        else:
            wvec = [[wscal(_WC_OFF + k * 16 + j) for k in range(F)]
                    for j in range(16)]
            bcv = [wscal(_BC_OFF + j) for j in range(16)]

        @pl.loop(0, RNG // LANES)
        def _(i):
            sl = pl.ds(i * LANES, LANES)
            gsl = pl.ds(rng0 + i * LANES, LANES)
            d = dis_rng[sl]
            t = [_tanh16((agg_rng[f, sl] + hs_v[f, gsl]) * d + bvec[f])
                 for f in range(F)]
            if not last:
                for j in range(f_out):
                    s = t[0] * wvec[j][0]
                    for k in range(1, F):
                        s = s + t[k] * wvec[j][k]
                    stage4[j, sl] = s * d
            else:
                lane = jnp.arange(LANES, dtype=jnp.int32)
                n2 = lane * 2 + i * (LANES * 2)
                n16 = lane * 16 + i * (LANES * 16)
                for f in range(F):
                    plsc.store_scatter(h3buf, [n2 + f], t[f])
                for j in range(16):
                    s = t[0] * wvec[j][0]
                    for k in range(1, F):
                        s = s + t[k] * wvec[j][k]
                    plsc.store_scatter(obuf, [n16 + j], s + bcv[j])

        if not last:
            pltpu.sync_copy(stage4.at[pl.ds(0, f_out)],
                            hs_sh.at[cid, pl.ds(0, f_out), pl.ds(rng0, RNG)])
            plsc.subcore_barrier()
            pltpu.sync_copy(hs_sh.at[cid, pl.ds(0, f_out)],
                            hs_v.at[pl.ds(0, f_out)])
            plsc.subcore_barrier()
        else:
            @pl.when(cid == 0)
            def _():
                o2 = pl.multiple_of(sid * (RNG * 2), 128)
                o16 = pl.multiple_of(sid * (RNG * 16), 128)
                pltpu.sync_copy(h3buf, h3_hbm.at[pl.ds(o2, RNG * 2)])
                pltpu.sync_copy(obuf, out_hbm.at[pl.ds(o16, RNG * 16)])

    layer(4, _W2_OFF, 4, _B1_OFF, False)
    layer(4, _W3_OFF, 2, _B2_OFF, False)
    layer(2, 0, 16, _B3_OFF, True)


def kernel(x, edge_index, W1, b1, W2, b2, W3, b3, Wc, bc):
    ei3 = edge_index.reshape(2, 16, EPT)
    h0t, pk = _tc_prep(x, W1, ei3)
    wt = jnp.concatenate([
        W2.ravel(), W3.ravel(), b1, b2, b3, Wc.ravel(), bc,
        jnp.zeros((_WT_LEN - 82,), jnp.float32),
    ])
    h3f, outf = _gcn_sc(h0t, pk, wt)
    return outf.reshape(NPAD, 16)[:N], h3f.reshape(NPAD, 2)[:N]


# speedup vs baseline: 1.3193x; 1.3138x over previous
import functools

import jax
import jax.numpy as jnp
from jax import lax
from jax.experimental import pallas as pl
from jax.experimental.pallas import tpu as pltpu
from jax.experimental.pallas import tpu_sc as plsc

N = 10000
NPAD = 10240
E = 320000
EPT = E // 16
LANES = 16
RNG = NPAD // 16
NCH = RNG // 128

_MESH = plsc.VectorSubcoreMesh(core_axis_name="c", subcore_axis_name="s")
_SC_PARAMS = pltpu.CompilerParams(needs_layout_passes=False)

_W2_OFF = 0
_W3_OFF = 16
_B1_OFF = 24
_B2_OFF = 28
_B3_OFF = 32
_WC_OFF = 34
_BC_OFF = 66
_WT_LEN = 96


def _tc_prep(x, W1, ei):
    def body(x_ref, w1_ref, ei_ref, h0_ref, pk_ref):
        h0t = lax.dot_general(
            w1_ref[...], x_ref[...], (((0,), (1,)), ((), ())),
            preferred_element_type=jnp.float32)
        h0_ref[...] = jnp.zeros_like(h0_ref)
        h0_ref[:, :N] = h0t
        pk_ref[...] = ei_ref[0] * 16384 + ei_ref[1]

    return pl.pallas_call(
        body,
        out_shape=(
            jax.ShapeDtypeStruct((4, NPAD), jnp.float32),
            jax.ShapeDtypeStruct((16, EPT), jnp.int32),
        ),
    )(x, W1, ei)


def _rsqrt16(d):
    i = plsc.bitcast(d, jnp.int32)
    i = 0x5F3759DF - lax.shift_right_logical(i, 1)
    y = plsc.bitcast(i, jnp.float32)
    for _ in range(3):
        y = y * (1.5 - 0.5 * d * y * y)
    return y


def _tanh16(z):
    e = jnp.exp(z + z)
    return 1.0 - 2.0 / (e + 1.0)


@functools.partial(
    pl.kernel,
    out_type=(
        jax.ShapeDtypeStruct((2, NPAD), jnp.float32),
        jax.ShapeDtypeStruct((16, NPAD), jnp.float32),
    ),
    mesh=_MESH,
    compiler_params=_SC_PARAMS,
    scratch_types=[
        pltpu.VMEM((4, NPAD), jnp.float32),
        pltpu.VMEM((4, NPAD), jnp.float32),
        pltpu.VMEM((EPT,), jnp.int32),
        pltpu.VMEM((_WT_LEN,), jnp.float32),
        pltpu.VMEM((4, RNG), jnp.float32),
        pltpu.VMEM((4, RNG), jnp.float32),
        pltpu.VMEM((RNG,), jnp.float32),
        pltpu.VMEM((16, 4, 128), jnp.float32),
        pltpu.VMEM((16, 128), jnp.float32),
        pltpu.VMEM((16, RNG), jnp.float32),
        pltpu.HBM((2, 16, 4, NPAD), jnp.float32),
        pltpu.VMEM_SHARED((4, NPAD), jnp.float32),
        pltpu.SemaphoreType.DMA,
        pltpu.SemaphoreType.DMA,
        pltpu.SemaphoreType.DMA,
    ],
)
def _gcn_sc(h0_hbm, pk_hbm, wt_hbm, h3_hbm, out_hbm, hs_v, acc_v, pk_v, wt_v,
            stage4, agg_rng, dis_rng, red_v, redd, obuf, slots, hs_sh,
            sem1, sem2, sem3):
    cid = lax.axis_index("c")
    sid = lax.axis_index("s")
    rng0 = pl.multiple_of(sid * RNG, 128)

    cp_pk = pltpu.async_copy(pk_hbm.at[sid], pk_v, sem1)
    cp_h0 = pltpu.async_copy(h0_hbm.at[:, pl.ds(rng0, RNG)], stage4, sem2)
    cp_wt = pltpu.async_copy(wt_hbm, wt_v, sem3)

    zero = jnp.zeros((LANES,), jnp.float32)
    ones = jnp.ones((LANES,), jnp.float32)
    fidx = [jnp.full((LANES,), f, jnp.int32) for f in range(4)]
    mask = jnp.full((LANES,), 16383, jnp.int32)

    @functools.partial(plsc.parallel_loop, 0, NPAD // LANES, unroll=8)
    def _(i):
        acc_v[0, pl.ds(i * LANES, LANES)] = zero

    cp_pk.wait()

    @functools.partial(plsc.parallel_loop, 0, EPT // LANES, unroll=4)
    def _(i):
        p = pk_v[pl.ds(i * LANES, LANES)]
        c = lax.bitwise_and(p, mask)
        plsc.addupdate_scatter(acc_v, [fidx[0], c], ones)

    pltpu.sync_copy(acc_v.at[0], slots.at[cid, sid, 0])
    plsc.subcore_barrier()

    @pl.loop(0, NCH)
    def _(ch):
        off = pl.multiple_of(rng0 + ch * 128, 128)
        pltpu.sync_copy(slots.at[cid, :, 0, pl.ds(off, 128)], redd)
        for i in range(8):
            sl = pl.ds(i * LANES, LANES)
            a = redd[0, sl]
            for k in range(1, 16):
                a = a + redd[k, sl]
            dis_rng[pl.ds(ch * 128 + i * LANES, LANES)] = _rsqrt16(a + 1.0)

    cp_h0.wait()
    cp_wt.wait()

    @pl.loop(0, RNG // LANES)
    def _(i):
        sl = pl.ds(i * LANES, LANES)
        d = dis_rng[sl]
        for f in range(4):
            stage4[f, sl] = stage4[f, sl] * d

    pltpu.sync_copy(stage4, hs_sh.at[:, pl.ds(rng0, RNG)])
    plsc.subcore_barrier()
    pltpu.sync_copy(hs_sh, hs_v)
    plsc.subcore_barrier()

    wchunk = [wt_v[pl.ds(16 * k, 16)] for k in range(_WT_LEN // 16)]

    def wscal(i):
        return jnp.broadcast_to(wchunk[i // 16][i % 16], (LANES,))

    def layer(F, w_off, f_out, b_off, last):
        @functools.partial(plsc.parallel_loop, 0, NPAD // LANES, unroll=8)
        def _(i):
            for f in range(F):
                acc_v[f, pl.ds(i * LANES, LANES)] = zero

        @functools.partial(plsc.parallel_loop, 0, EPT // LANES, unroll=4)
        def _(i):
            p = pk_v[pl.ds(i * LANES, LANES)]
            r = lax.shift_right_logical(p, 14)
            c = lax.bitwise_and(p, mask)
            for f in range(F):
                v = plsc.load_gather(hs_v, [fidx[f], r])
                plsc.addupdate_scatter(acc_v, [fidx[f], c], v)

        pltpu.sync_copy(acc_v.at[pl.ds(0, F)], slots.at[cid, sid, pl.ds(0, F)])
        plsc.subcore_barrier()

        @pl.loop(0, NCH)
        def _(ch):
            off = pl.multiple_of(rng0 + ch * 128, 128)
            pltpu.sync_copy(slots.at[cid, :, pl.ds(0, F), pl.ds(off, 128)],
                            red_v.at[:, pl.ds(0, F)])
            for i in range(8):
                sl = pl.ds(i * LANES, LANES)
                dst = pl.ds(ch * 128 + i * LANES, LANES)
                for f in range(F):
                    a = red_v[0, f, sl]
                    for k in range(1, 16):
                        a = a + red_v[k, f, sl]
                    agg_rng[f, dst] = a

        bvec = [wscal(b_off + f) for f in range(F)]
        if not last:
            wvec = [[wscal(w_off + k * f_out + j) for k in range(F)]
                    for j in range(f_out)]
        else:
            wvec = [[wscal(_WC_OFF + k * 16 + j) for k in range(F)]
                    for j in range(16)]
            bcv = [wscal(_BC_OFF + j) for j in range(16)]

        @pl.loop(0, RNG // LANES)
        def _(i):
            sl = pl.ds(i * LANES, LANES)
            gsl = pl.ds(rng0 + i * LANES, LANES)
            d = dis_rng[sl]
            t = [_tanh16((agg_rng[f, sl] + hs_v[f, gsl]) * d + bvec[f])
                 for f in range(F)]
            if not last:
                for j in range(f_out):
                    s = t[0] * wvec[j][0]
                    for k in range(1, F):
                        s = s + t[k] * wvec[j][k]
                    stage4[j, sl] = s * d
            else:
                for f in range(F):
                    stage4[f, sl] = t[f]
                for j in range(16):
                    s = t[0] * wvec[j][0]
                    for k in range(1, F):
                        s = s + t[k] * wvec[j][k]
                    obuf[j, sl] = s + bcv[j]

        if not last:
            pltpu.sync_copy(stage4.at[pl.ds(0, f_out)],
                            hs_sh.at[pl.ds(0, f_out), pl.ds(rng0, RNG)])
            plsc.subcore_barrier()
            pltpu.sync_copy(hs_sh.at[pl.ds(0, f_out)],
                            hs_v.at[pl.ds(0, f_out)])
            plsc.subcore_barrier()
        else:
            @pl.when(cid == 0)
            def _():
                pltpu.sync_copy(stage4.at[pl.ds(0, 2)],
                                h3_hbm.at[:, pl.ds(rng0, RNG)])
                pltpu.sync_copy(obuf, out_hbm.at[:, pl.ds(rng0, RNG)])

    layer(4, _W2_OFF, 4, _B1_OFF, False)
    layer(4, _W3_OFF, 2, _B2_OFF, False)
    layer(2, 0, 16, _B3_OFF, True)


def kernel(x, edge_index, W1, b1, W2, b2, W3, b3, Wc, bc):
    ei3 = edge_index.reshape(2, 16, EPT)
    h0t, pk = _tc_prep(x, W1, ei3)
    wt = jnp.concatenate([
        W2.ravel(), W3.ravel(), b1, b2, b3, Wc.ravel(), bc,
        jnp.zeros((_WT_LEN - 82,), jnp.float32),
    ])
    h3t, outt = _gcn_sc(h0t, pk, wt)
    return outt[:, :N].T, h3t[:, :N].T


# speedup vs baseline: 1.3292x; 1.0075x over previous
import functools

import jax
import jax.numpy as jnp
from jax import lax
from jax.experimental import pallas as pl
from jax.experimental.pallas import tpu as pltpu
from jax.experimental.pallas import tpu_sc as plsc

N = 10000
NPAD = 10240
E = 320000
EPT = E // 16
LANES = 16
RNG = NPAD // 16
NCH = RNG // 128

_MESH = plsc.VectorSubcoreMesh(core_axis_name="c", subcore_axis_name="s")
_SC_PARAMS = pltpu.CompilerParams(needs_layout_passes=False)

_W2_OFF = 0
_W3_OFF = 16
_B1_OFF = 24
_B2_OFF = 28
_B3_OFF = 32
_WC_OFF = 34
_BC_OFF = 66
_WT_LEN = 96


def _tc_prep(x, W1, ei):
    def body(x_ref, w1_ref, ei_ref, h0_ref, pk_ref):
        h0t = lax.dot_general(
            w1_ref[...], x_ref[...], (((0,), (1,)), ((), ())),
            preferred_element_type=jnp.float32)
        h0_ref[...] = jnp.zeros_like(h0_ref)
        h0_ref[:, :N] = h0t
        pk_ref[...] = ei_ref[0] * 16384 + ei_ref[1]

    return pl.pallas_call(
        body,
        out_shape=(
            jax.ShapeDtypeStruct((4, NPAD), jnp.float32),
            jax.ShapeDtypeStruct((16, EPT), jnp.int32),
        ),
    )(x, W1, ei)


def _rsqrt16(d):
    i = plsc.bitcast(d, jnp.int32)
    i = 0x5F3759DF - lax.shift_right_logical(i, 1)
    y = plsc.bitcast(i, jnp.float32)
    for _ in range(3):
        y = y * (1.5 - 0.5 * d * y * y)
    return y


def _tanh16(z):
    e = jnp.exp(z + z)
    return 1.0 - 2.0 / (e + 1.0)


@functools.partial(
    pl.kernel,
    out_type=(
        jax.ShapeDtypeStruct((2, NPAD), jnp.float32),
        jax.ShapeDtypeStruct((16, NPAD), jnp.float32),
    ),
    mesh=_MESH,
    compiler_params=_SC_PARAMS,
    scratch_types=[
        pltpu.VMEM((4, NPAD), jnp.float32),
        pltpu.VMEM((4, NPAD), jnp.float32),
        pltpu.VMEM((EPT,), jnp.int32),
        pltpu.VMEM((_WT_LEN,), jnp.float32),
        pltpu.VMEM((4, RNG), jnp.float32),
        pltpu.VMEM((4, RNG), jnp.float32),
        pltpu.VMEM((RNG,), jnp.float32),
        pltpu.VMEM((16, 4, 128), jnp.float32),
        pltpu.VMEM((16, 128), jnp.float32),
        pltpu.VMEM((16, RNG), jnp.float32),
        pltpu.HBM((2, 16, 4, NPAD), jnp.float32),
        pltpu.VMEM_SHARED((4, NPAD), jnp.float32),
        pltpu.SemaphoreType.DMA,
        pltpu.SemaphoreType.DMA,
        pltpu.SemaphoreType.DMA,
    ],
)
def _gcn_sc(h0_hbm, pk_hbm, wt_hbm, h3_hbm, out_hbm, hs_v, acc_v, pk_v, wt_v,
            stage4, agg_rng, dis_rng, red_v, redd, obuf, slots, hs_sh,
            sem1, sem2, sem3):
    cid = lax.axis_index("c")
    sid = lax.axis_index("s")
    rng0 = pl.multiple_of(sid * RNG, 128)

    cp_pk = pltpu.async_copy(pk_hbm.at[sid], pk_v, sem1)
    cp_h0 = pltpu.async_copy(h0_hbm.at[:, pl.ds(rng0, RNG)], stage4, sem2)
    cp_wt = pltpu.async_copy(wt_hbm, wt_v, sem3)

    zero = jnp.zeros((LANES,), jnp.float32)
    ones = jnp.ones((LANES,), jnp.float32)
    fidx = [jnp.full((LANES,), f, jnp.int32) for f in range(4)]
    mask = jnp.full((LANES,), 16383, jnp.int32)

    @functools.partial(plsc.parallel_loop, 0, NPAD // LANES, unroll=8)
    def _(i):
        acc_v[0, pl.ds(i * LANES, LANES)] = zero

    cp_pk.wait()

    @functools.partial(plsc.parallel_loop, 0, EPT // LANES, unroll=4)
    def _(i):
        p = pk_v[pl.ds(i * LANES, LANES)]
        c = lax.bitwise_and(p, mask)
        plsc.addupdate_scatter(acc_v, [fidx[0], c], ones)

    pltpu.sync_copy(acc_v.at[0], slots.at[cid, sid, 0])

    @functools.partial(plsc.parallel_loop, 0, NPAD // LANES, unroll=8)
    def _(i):
        for f in range(4):
            acc_v[f, pl.ds(i * LANES, LANES)] = zero

    plsc.subcore_barrier()

    @pl.loop(0, NCH)
    def _(ch):
        off = pl.multiple_of(rng0 + ch * 128, 128)
        pltpu.sync_copy(slots.at[cid, :, 0, pl.ds(off, 128)], redd)
        for i in range(8):
            sl = pl.ds(i * LANES, LANES)
            a = redd[0, sl]
            for k in range(1, 16):
                a = a + redd[k, sl]
            dis_rng[pl.ds(ch * 128 + i * LANES, LANES)] = _rsqrt16(a + 1.0)

    cp_h0.wait()
    cp_wt.wait()

    @pl.loop(0, RNG // LANES)
    def _(i):
        sl = pl.ds(i * LANES, LANES)
        d = dis_rng[sl]
        for f in range(4):
            stage4[f, sl] = stage4[f, sl] * d

    pltpu.sync_copy(stage4, hs_sh.at[:, pl.ds(rng0, RNG)])
    plsc.subcore_barrier()
    pltpu.sync_copy(hs_sh, hs_v)
    plsc.subcore_barrier()

    wchunk = [wt_v[pl.ds(16 * k, 16)] for k in range(_WT_LEN // 16)]

    def wscal(i):
        return jnp.broadcast_to(wchunk[i // 16][i % 16], (LANES,))

    def layer(F, w_off, f_out, b_off, last):
        @functools.partial(plsc.parallel_loop, 0, EPT // LANES, unroll=4)
        def _(i):
            p = pk_v[pl.ds(i * LANES, LANES)]
            r = lax.shift_right_logical(p, 14)
            c = lax.bitwise_and(p, mask)
            for f in range(F):
                v = plsc.load_gather(hs_v, [fidx[f], r])
                plsc.addupdate_scatter(acc_v, [fidx[f], c], v)

        pltpu.sync_copy(acc_v.at[pl.ds(0, F)], slots.at[cid, sid, pl.ds(0, F)])

        if not last:
            @functools.partial(plsc.parallel_loop, 0, NPAD // LANES, unroll=8)
            def _(i):
                for f in range(F):
                    acc_v[f, pl.ds(i * LANES, LANES)] = zero

        plsc.subcore_barrier()

        @pl.loop(0, NCH)
        def _(ch):
            off = pl.multiple_of(rng0 + ch * 128, 128)
            pltpu.sync_copy(slots.at[cid, :, pl.ds(0, F), pl.ds(off, 128)],
                            red_v.at[:, pl.ds(0, F)])
            for i in range(8):
                sl = pl.ds(i * LANES, LANES)
                dst = pl.ds(ch * 128 + i * LANES, LANES)
                for f in range(F):
                    a = red_v[0, f, sl]
                    for k in range(1, 16):
                        a = a + red_v[k, f, sl]
                    agg_rng[f, dst] = a

        bvec = [wscal(b_off + f) for f in range(F)]
        if not last:
            wvec = [[wscal(w_off + k * f_out + j) for k in range(F)]
                    for j in range(f_out)]
        else:
            wvec = [[wscal(_WC_OFF + k * 16 + j) for k in range(F)]
                    for j in range(16)]
            bcv = [wscal(_BC_OFF + j) for j in range(16)]

        @pl.loop(0, RNG // LANES)
        def _(i):
            sl = pl.ds(i * LANES, LANES)
            gsl = pl.ds(rng0 + i * LANES, LANES)
            d = dis_rng[sl]
            t = [_tanh16((agg_rng[f, sl] + hs_v[f, gsl]) * d + bvec[f])
                 for f in range(F)]
            if not last:
                for j in range(f_out):
                    s = t[0] * wvec[j][0]
                    for k in range(1, F):
                        s = s + t[k] * wvec[j][k]
                    stage4[j, sl] = s * d
            else:
                for f in range(F):
                    stage4[f, sl] = t[f]
                for j in range(16):
                    s = t[0] * wvec[j][0]
                    for k in range(1, F):
                        s = s + t[k] * wvec[j][k]
                    obuf[j, sl] = s + bcv[j]

        if not last:
            pltpu.sync_copy(stage4.at[pl.ds(0, f_out)],
                            hs_sh.at[pl.ds(0, f_out), pl.ds(rng0, RNG)])
            plsc.subcore_barrier()
            pltpu.sync_copy(hs_sh.at[pl.ds(0, f_out)],
                            hs_v.at[pl.ds(0, f_out)])
            plsc.subcore_barrier()
        else:
            @pl.when(cid == 0)
            def _():
                pltpu.sync_copy(stage4.at[pl.ds(0, 2)],
                                h3_hbm.at[:, pl.ds(rng0, RNG)])
                pltpu.sync_copy(obuf, out_hbm.at[:, pl.ds(rng0, RNG)])

    layer(4, _W2_OFF, 4, _B1_OFF, False)
    layer(4, _W3_OFF, 2, _B2_OFF, False)
    layer(2, 0, 16, _B3_OFF, True)


def kernel(x, edge_index, W1, b1, W2, b2, W3, b3, Wc, bc):
    ei3 = edge_index.reshape(2, 16, EPT)
    h0t, pk = _tc_prep(x, W1, ei3)
    wt = jnp.concatenate([
        W2.ravel(), W3.ravel(), b1, b2, b3, Wc.ravel(), bc,
        jnp.zeros((_WT_LEN - 82,), jnp.float32),
    ])
    h3t, outt = _gcn_sc(h0t, pk, wt)
    return outt[:, :N].T, h3t[:, :N].T
